# Initial kernel scaffold; baseline (speedup 1.0000x reference)
import jax
import jax.numpy as jnp
from jax.experimental import pallas as pl


def kernel(x, edge_index, batch, params):
    raise NotImplementedError("write your pallas kernel here")


# speedup vs baseline: 5.0535x; 5.0535x over previous
import functools

import jax
import jax.numpy as jnp
from jax import lax
from jax.experimental import pallas as pl
from jax.experimental.compute_on import compute_on
from jax.experimental.pallas import tpu as pltpu
from jax.experimental.pallas import tpu_sc as plsc

N = 10000
E = 320000
D = 128
G = 256
NUM_LAYERS = 8

NC = 2
NS = 16
NW = NC * NS
PE = E // NW
CH = 128
NFULL = PE // CH
TAIL = PE - NFULL * CH
NROWCH = N // CH
ROWTAIL = N - NROWCH * CH
SEGW = G // NW

_mesh = plsc.VectorSubcoreMesh(core_axis_name="c", subcore_axis_name="s",
                               num_cores=NC, num_subcores=NS)


def _on_sc(fn):
    wrapped = compute_on("tpu_sparsecore")(fn)
    return wrapped


@functools.partial(
    pl.kernel,
    out_type=(jax.ShapeDtypeStruct((NC, N, D), jnp.float32),
              jax.ShapeDtypeStruct((G, D), jnp.float32)),
    mesh=_mesh,
    scratch_types=[
        pltpu.VMEM((CH, D), jnp.float32),
        pltpu.VMEM((CH,), jnp.int32),
        pltpu.VMEM((CH,), jnp.int32),
        pltpu.VMEM((TAIL, D), jnp.float32),
        pltpu.VMEM((TAIL,), jnp.int32),
        pltpu.VMEM((TAIL,), jnp.int32),
        pltpu.VMEM((N + 16,), jnp.int32),
        pltpu.VMEM((SEGW, D), jnp.float32),
        pltpu.VMEM_SHARED((N, D), jnp.float32),
        pltpu.SemaphoreType.DMA,
    ],
)
def _layer_sc(h_hbm, src_hbm, dst_hbm, batch_hbm, out_hbm, pool_hbm,
              rows_v, sidx_v, didx_v, rows_t, sidx_t, didx_t,
              batch_v, macc_v, acc, sem):
    cid = lax.axis_index("c")
    sid = lax.axis_index("s")
    wid = cid * NS + sid

    def _zrow(i, _):
        def _zcol(j, _):
            rows_v[i, pl.ds(j * 16, 16)] = jnp.zeros((16,), jnp.float32)
            return _
        return lax.fori_loop(0, D // 16, _zcol, _)
    lax.fori_loop(0, CH, _zrow, None)

    for k in range((NROWCH + NS - 1) // NS):
        c = sid + k * NS

        @pl.when(c < NROWCH)
        def _():
            pltpu.sync_copy(rows_v, acc.at[pl.ds(c * CH, CH)])

    @pl.when(sid == NS - 1)
    def _():
        pltpu.sync_copy(rows_v.at[pl.ds(0, ROWTAIL)],
                        acc.at[pl.ds(NROWCH * CH, ROWTAIL)])

    plsc.subcore_barrier()

    ebase = wid * PE

    def _body(k, _):
        off = ebase + k * CH
        pltpu.sync_copy(src_hbm.at[pl.ds(off, CH)], sidx_v)
        pltpu.sync_copy(dst_hbm.at[pl.ds(off, CH)], didx_v)
        pltpu.async_copy(h_hbm.at[sidx_v], rows_v, sem).wait()
        pltpu.sync_copy(rows_v, acc.at[didx_v], add=True)
        return _
    lax.fori_loop(0, NFULL, _body, None)

    toff = ebase + NFULL * CH
    pltpu.sync_copy(src_hbm.at[pl.ds(toff, TAIL)], sidx_t)
    pltpu.sync_copy(dst_hbm.at[pl.ds(toff, TAIL)], didx_t)
    pltpu.async_copy(h_hbm.at[sidx_t], rows_t, sem).wait()
    pltpu.sync_copy(rows_t, acc.at[didx_t], add=True)

    plsc.subcore_barrier()

    for k in range((NROWCH + NS - 1) // NS):
        c = sid + k * NS

        @pl.when(c < NROWCH)
        def _():
            pltpu.sync_copy(acc.at[pl.ds(c * CH, CH)],
                            out_hbm.at[cid, pl.ds(c * CH, CH)])

    @pl.when(sid == NS - 1)
    def _():
        pltpu.sync_copy(acc.at[pl.ds(NROWCH * CH, ROWTAIL)],
                        out_hbm.at[cid, pl.ds(NROWCH * CH, ROWTAIL)])

    _segmax_work(h_hbm, batch_hbm, pool_hbm, batch_v, rows_v, macc_v, wid)


def _segmax_work(h_hbm, batch_hbm, out_hbm, batch_v, rows_v, acc_v, wid):
    g0 = wid * SEGW

    pltpu.sync_copy(batch_hbm, batch_v.at[pl.ds(0, N)])
    batch_v[pl.ds(N, 16)] = jnp.full((16,), G, jnp.int32)

    def _cnt(k, carry):
        lo, hi = carry
        b16 = batch_v[pl.ds(k * 16, 16)]
        lo = lo + jnp.where(b16 < g0, 1, 0).astype(jnp.int32)
        hi = hi + jnp.where(b16 < g0 + SEGW, 1, 0).astype(jnp.int32)
        return lo, hi
    z16 = jnp.zeros((16,), jnp.int32)
    lo, hi = lax.fori_loop(0, N // 16, _cnt, (z16, z16))

    def _lanesum(v):
        s = v[0]
        for i in range(1, 16):
            s = s + v[i]
        return s
    start = _lanesum(lo)
    end = _lanesum(hi)

    for i in range(SEGW):
        for j in range(D // 16):
            acc_v[i, pl.ds(j * 16, 16)] = jnp.full((16,), -jnp.inf, jnp.float32)

    astart = start - lax.rem(start, 8)
    nch = (end - astart + CH - 1) // CH

    def _chunk(k, _):
        off = pl.multiple_of(jnp.minimum(astart + k * CH, N - CH), 8)
        pltpu.sync_copy(h_hbm.at[pl.ds(off, CH)], rows_v)

        def _row(r, _):
            seg = batch_v[pl.ds(off + r, 16)][0]
            ok = jnp.logical_and(seg >= g0, seg < g0 + SEGW)

            @pl.when(ok)
            def _():
                loc = seg - g0
                for j in range(D // 16):
                    cur = acc_v[loc, pl.ds(j * 16, 16)]
                    acc_v[loc, pl.ds(j * 16, 16)] = jnp.maximum(
                        cur, rows_v[r, pl.ds(j * 16, 16)])
            return _
        return lax.fori_loop(0, CH, _row, None)
    lax.fori_loop(0, nch, _chunk, None)

    pltpu.sync_copy(acc_v, out_hbm.at[pl.ds(g0, SEGW)])


@functools.partial(
    pl.kernel,
    out_type=jax.ShapeDtypeStruct((G, D), jnp.float32),
    mesh=_mesh,
    scratch_types=[
        pltpu.VMEM((N + 16,), jnp.int32),
        pltpu.VMEM((CH, D), jnp.float32),
        pltpu.VMEM((SEGW, D), jnp.float32),
    ],
)
def _segmax(h_hbm, batch_hbm, out_hbm, batch_v, rows_v, acc_v):
    cid = lax.axis_index("c")
    sid = lax.axis_index("s")
    wid = cid * NS + sid
    _segmax_work(h_hbm, batch_hbm, out_hbm, batch_v, rows_v, acc_v, wid)


def _lin(t, w_ref, b_ref):
    z = lax.dot_general(t, w_ref[...], (((1,), (1,)), ((), ())),
                        preferred_element_type=jnp.float32)
    return z + b_ref[...]


def _bn(z, g_ref, b_ref):
    m = jnp.mean(z, axis=0, keepdims=True)
    v = jnp.mean((z - m) ** 2, axis=0, keepdims=True)
    return g_ref[...] * (z - m) / jnp.sqrt(v + 1e-5) + b_ref[...]


def _mlp3(q, w0, b0, w1, b1, w2, b2):
    q = jnp.maximum(_lin(q, w0, b0), 0.0)
    q = jnp.maximum(_lin(q, w1, b1), 0.0)
    return _lin(q, w2, b2)


def _dense_body(h_ref, p_ref, pool_ref, sin_ref,
                w1, b1, g1, be1, w2, b2, g2, be2, w3, b3, go, beo,
                sw0, sb0, sw1, sb1, sw2, sb2, out_ref, sout_ref):
    t = h_ref[...] + p_ref[0] + p_ref[1]
    t = jnp.maximum(_bn(_lin(t, w1, b1), g1, be1), 0.0)
    t = jnp.maximum(_bn(_lin(t, w2, b2), g2, be2), 0.0)
    t = jnp.maximum(_bn(_lin(t, w3, b3), go, beo), 0.0)
    out_ref[...] = t
    sout_ref[...] = sin_ref[...] + _mlp3(pool_ref[...],
                                         sw0, sb0, sw1, sb1, sw2, sb2)


_dense = pl.pallas_call(
    _dense_body,
    out_shape=(jax.ShapeDtypeStruct((N, D), jnp.float32),
               jax.ShapeDtypeStruct((G, D), jnp.float32)),
)


def _scorefin_body(pool_ref, sin_ref, sw0, sb0, sw1, sb1, sw2, sb2, out_ref):
    out_ref[...] = sin_ref[...] + _mlp3(pool_ref[...],
                                        sw0, sb0, sw1, sb1, sw2, sb2)


_scorefin = pl.pallas_call(
    _scorefin_body,
    out_shape=jax.ShapeDtypeStruct((G, D), jnp.float32),
)


def kernel(x, edge_index, batch, params):
    src = edge_index[0]
    dst = edge_index[1]
    h = x
    score = jnp.zeros((G, D), jnp.float32)
    layer_sc = _on_sc(_layer_sc)
    segmax_sc = _on_sc(_segmax)
    for i in range(NUM_LAYERS - 1):
        parts, pool_i = layer_sc(h, src, dst, batch)
        c = params["convs"][i]
        (w1, b1), (w2, b2), (w3, b3) = c["lins"]
        (g1, be1), (g2, be2) = c["bns"]
        go, beo = params["outer_bns"][i]
        (sw0, sb0), (sw1, sb1), (sw2, sb2) = params["score_mlps"][i]
        h, score = _dense(h, parts, pool_i, score,
                          w1, b1, g1, be1, w2, b2, g2, be2, w3, b3, go, beo,
                          sw0, sb0, sw1, sb1, sw2, sb2)
    pool_f = segmax_sc(h, batch)
    (sw0, sb0), (sw1, sb1), (sw2, sb2) = params["score_mlps"][NUM_LAYERS - 1]
    return _scorefin(pool_f, score, sw0, sb0, sw1, sb1, sw2, sb2)
